# baseline (speedup 1.0000x reference)
import functools

import jax
import jax.numpy as jnp
from jax import lax
from jax.experimental import pallas as pl
from jax.experimental.pallas import tpu as pltpu
from jax.experimental.pallas import tpu_sc as plsc

_DIM = 64
_L = 16384
_NC = 2
_NS = 16
_NW = _NC * _NS
_CHUNK = _L // _NW
_GSUB = 128
_NGATHER = _CHUNK // _GSUB
_LANES = 16
_DREG = _DIM // _LANES


def _weights_body(g_ref, c_ref, w_ref):
    g = g_ref[...]
    c = c_ref[...]
    t = jnp.log1p(jnp.where(g >= 0, c, 0.0))
    w_ref[...] = t * (1.0 / jnp.sum(t))


_weights = pl.pallas_call(
    _weights_body,
    out_shape=jax.ShapeDtypeStruct((128, 128), jnp.float32),
)

_mesh = plsc.VectorSubcoreMesh(
    core_axis_name="c", subcore_axis_name="s", num_cores=_NC, num_subcores=_NS
)


@functools.partial(
    pl.kernel,
    out_type=jax.ShapeDtypeStruct((_NW, _DIM), jnp.float32),
    mesh=_mesh,
    compiler_params=pltpu.CompilerParams(use_tc_tiling_on_sc=False),
    scratch_types=[
        pltpu.VMEM((_NGATHER, _GSUB), jnp.int32),
        pltpu.VMEM((_CHUNK,), jnp.float32),
        pltpu.VMEM((_CHUNK, _DIM), jnp.float32),
        pltpu.VMEM((_DIM,), jnp.float32),
        pltpu.SemaphoreType.DMA,
    ],
)
def _sc_embed(gids_hbm, w_hbm, table_hbm, out_hbm, idx_v, w_v, rows_v, acc_v, sem):
    wid = lax.axis_index("s") * _NC + lax.axis_index("c")

    pltpu.sync_copy(gids_hbm.at[wid], idx_v)
    pltpu.sync_copy(w_hbm.at[wid], w_v)

    copies = [
        pltpu.async_copy(
            table_hbm.at[idx_v.at[j]],
            rows_v.at[pl.ds(j * _GSUB, _GSUB)],
            sem,
        )
        for j in range(_NGATHER)
    ]
    for c in copies:
        c.wait()

    zero = jnp.zeros((_LANES,), jnp.float32)

    def body(k, carry):
        wvec = w_v[pl.ds(k * _LANES, _LANES)]
        acc = carry
        for j in range(_LANES):
            i = k * _LANES + j
            wi = wvec[j]
            acc = tuple(
                acc[d] + wi * rows_v[i, pl.ds(d * _LANES, _LANES)]
                for d in range(_DREG)
            )
        return acc

    acc = lax.fori_loop(0, _CHUNK // _LANES, body, (zero,) * _DREG)
    for d in range(_DREG):
        acc_v[pl.ds(d * _LANES, _LANES)] = acc[d]
    pltpu.sync_copy(acc_v, out_hbm.at[wid])


@jax.jit
def kernel(gids, cnts, table):
    gids = gids.astype(jnp.int32)
    w2d = _weights(gids.reshape(128, 128), cnts.reshape(128, 128))
    partials = _sc_embed(
        gids.reshape(_NW, _NGATHER, _GSUB),
        w2d.reshape(_NW, _CHUNK),
        table,
    )
    return partials.sum(axis=0)


# speedup vs baseline: 4.8112x; 4.8112x over previous
import functools

import jax
import jax.numpy as jnp
from jax import lax
from jax.experimental import pallas as pl
from jax.experimental.pallas import tpu as pltpu
from jax.experimental.pallas import tpu_sc as plsc

_DIM = 64
_L = 16384
_NG = 1000000
_NC = 2
_NS = 16
_NW = _NC * _NS
_CHUNK = _L // _NW
_LANES = 16
_NROW = _NG // _LANES
_ZROW = 3904
_ZREM = _NROW - _NS * _ZROW
_ALIGNED = 999936
_C = 27776
_NSTEP = _ALIGNED // _C
_ROWW = 128
_NROW2 = 7816
_NGPAD = _NROW2 * _ROWW
_ZROW2 = 488
_ZREM2 = _NROW2 - _NS * _ZROW2


def _weights_body(g_ref, c_ref, w_ref):
    g = g_ref[...]
    c = c_ref[...]
    t = jnp.log1p(jnp.where(g >= 0, c, 0.0))
    w_ref[...] = t * (1.0 / jnp.sum(t))


_weights = pl.pallas_call(
    _weights_body,
    out_shape=jax.ShapeDtypeStruct((128, 128), jnp.float32),
)

_mesh = plsc.VectorSubcoreMesh(
    core_axis_name="c", subcore_axis_name="s", num_cores=_NC, num_subcores=_NS
)


@functools.partial(
    pl.kernel,
    out_type=jax.ShapeDtypeStruct((_NC, _NROW2, _ROWW), jnp.float32),
    mesh=_mesh,
    compiler_params=pltpu.CompilerParams(needs_layout_passes=False),
    scratch_types=[
        pltpu.VMEM((_CHUNK,), jnp.int32),
        pltpu.VMEM((_CHUNK,), jnp.float32),
        pltpu.VMEM((_CHUNK,), jnp.int32),
        pltpu.VMEM((_CHUNK, _ROWW), jnp.float32),
        pltpu.VMEM_SHARED((_NROW2, _ROWW), jnp.float32),
    ],
)
def _sc_scatter(gids_hbm, w_hbm, dense_hbm,
                idx_v, w_v, ci_v, contrib_v, shared):
    core = lax.axis_index("c")
    sub = lax.axis_index("s")
    wid = sub * _NC + core

    pltpu.sync_copy(gids_hbm.at[wid], idx_v)
    pltpu.sync_copy(w_hbm.at[wid], w_v)

    lane = lax.iota(jnp.int32, _LANES)
    zvec = jnp.zeros((_LANES,), jnp.float32)

    def zfill(r, _):
        for c in range(_ROWW // _LANES):
            contrib_v[r, pl.ds(c * _LANES, _LANES)] = zvec
        return 0

    lax.fori_loop(0, _CHUNK, zfill, 0)

    pltpu.sync_copy(
        contrib_v.at[pl.ds(0, _ZROW2)],
        shared.at[pl.ds(sub * _ZROW2, _ZROW2)],
    )

    @pl.when(sub == 0)
    def _():
        pltpu.sync_copy(
            contrib_v.at[pl.ds(0, _ZREM2)],
            shared.at[pl.ds(_NS * _ZROW2, _ZREM2)],
        )

    def build(b, _):
        kbase = b * _LANES
        gvec = idx_v[pl.ds(kbase, _LANES)]
        ci_v[pl.ds(kbase, _LANES)] = lax.shift_right_logical(gvec, 7)
        wvec = w_v[pl.ds(kbase, _LANES)]
        rows = kbase + lane
        cols = gvec & (_ROWW - 1)
        plsc.store_scatter(contrib_v, [rows, cols], wvec)
        return 0

    lax.fori_loop(0, _CHUNK // _LANES, build, 0)

    plsc.subcore_barrier()
    pltpu.sync_copy(contrib_v, shared.at[ci_v], add=True)
    plsc.subcore_barrier()

    pltpu.sync_copy(
        shared.at[pl.ds(sub * _ZROW2, _ZROW2)],
        dense_hbm.at[core, pl.ds(sub * _ZROW2, _ZROW2)],
    )

    @pl.when(sub == 0)
    def _():
        pltpu.sync_copy(
            shared.at[pl.ds(_NS * _ZROW2, _ZREM2)],
            dense_hbm.at[core, pl.ds(_NS * _ZROW2, _ZREM2)],
        )


def _mv_body(w_ref, t_ref, o_ref):
    wsum = w_ref[0, :] + w_ref[1, :]

    @pl.when(pl.program_id(0) == 0)
    def _():
        o_ref[...] = jnp.zeros_like(o_ref)

    o_ref[...] += jax.lax.dot_general(
        t_ref[...], wsum,
        dimension_numbers=(((1,), (0,)), ((), ())),
        preferred_element_type=jnp.float32,
    )


_matvec = pl.pallas_call(
    _mv_body,
    grid=(_NSTEP,),
    in_specs=[
        pl.BlockSpec((_NC, _C), lambda i: (0, i)),
        pl.BlockSpec((_DIM, _C), lambda i: (0, i)),
    ],
    out_specs=pl.BlockSpec((_DIM,), lambda i: (0,)),
    out_shape=jax.ShapeDtypeStruct((_DIM,), jnp.float32),
)


def _mv_tail_body(w_ref, t_ref, o_ref):
    wsum = w_ref[0, :] + w_ref[1, :]
    o_ref[...] = jax.lax.dot_general(
        t_ref[...], wsum,
        dimension_numbers=(((1,), (0,)), ((), ())),
        preferred_element_type=jnp.float32,
    )


_mv_tail = pl.pallas_call(
    _mv_tail_body,
    out_shape=jax.ShapeDtypeStruct((_DIM,), jnp.float32),
)


@jax.jit
def kernel(gids, cnts, table):
    gids = gids.astype(jnp.int32)
    w2d = _weights(gids.reshape(128, 128), cnts.reshape(128, 128))
    dense = _sc_scatter(
        gids.reshape(_NW, _CHUNK),
        w2d.reshape(_NW, _CHUNK),
    ).reshape(_NC, _NGPAD)
    tt = table.T
    out_main = _matvec(dense, tt)
    out_tail = _mv_tail(
        lax.slice(dense, (0, _ALIGNED), (_NC, _NG)),
        lax.slice(tt, (0, _ALIGNED), (_DIM, _NG)),
    )
    return out_main + out_tail


# speedup vs baseline: 4.8179x; 1.0014x over previous
import functools

import jax
import jax.numpy as jnp
from jax import lax
from jax.experimental import pallas as pl
from jax.experimental.pallas import tpu as pltpu
from jax.experimental.pallas import tpu_sc as plsc

_DIM = 64
_L = 16384
_NG = 1000000
_NC = 2
_NS = 16
_NW = _NC * _NS
_CHUNK = _L // _NW
_LANES = 16
_NROW = _NG // _LANES
_ZROW = 3904
_ZREM = _NROW - _NS * _ZROW
_ALIGNED = 999936
_C = 55552
_NSTEP = _ALIGNED // _C
_ROWW = 128
_NROW2 = 7816
_NGPAD = _NROW2 * _ROWW
_ZROW2 = 488
_ZREM2 = _NROW2 - _NS * _ZROW2


def _weights_body(g_ref, c_ref, w_ref):
    g = g_ref[...]
    c = c_ref[...]
    t = jnp.log1p(jnp.where(g >= 0, c, 0.0))
    w_ref[...] = t * (1.0 / jnp.sum(t))


_weights = pl.pallas_call(
    _weights_body,
    out_shape=jax.ShapeDtypeStruct((128, 128), jnp.float32),
)

_mesh = plsc.VectorSubcoreMesh(
    core_axis_name="c", subcore_axis_name="s", num_cores=_NC, num_subcores=_NS
)


@functools.partial(
    pl.kernel,
    out_type=jax.ShapeDtypeStruct((_NC, _NROW2, _ROWW), jnp.float32),
    mesh=_mesh,
    compiler_params=pltpu.CompilerParams(needs_layout_passes=False),
    scratch_types=[
        pltpu.VMEM((_CHUNK,), jnp.int32),
        pltpu.VMEM((_CHUNK,), jnp.float32),
        pltpu.VMEM((_CHUNK,), jnp.int32),
        pltpu.VMEM((_CHUNK, _ROWW), jnp.float32),
        pltpu.VMEM_SHARED((_NROW2, _ROWW), jnp.float32),
    ],
)
def _sc_scatter(gids_hbm, w_hbm, dense_hbm,
                idx_v, w_v, ci_v, contrib_v, shared):
    core = lax.axis_index("c")
    sub = lax.axis_index("s")
    wid = sub * _NC + core

    pltpu.sync_copy(gids_hbm.at[wid], idx_v)
    pltpu.sync_copy(w_hbm.at[wid], w_v)

    lane = lax.iota(jnp.int32, _LANES)
    zvec = jnp.zeros((_LANES,), jnp.float32)

    def zfill(r, _):
        for c in range(_ROWW // _LANES):
            contrib_v[r, pl.ds(c * _LANES, _LANES)] = zvec
        return 0

    lax.fori_loop(0, _CHUNK, zfill, 0)

    pltpu.sync_copy(
        contrib_v.at[pl.ds(0, _ZROW2)],
        shared.at[pl.ds(sub * _ZROW2, _ZROW2)],
    )

    @pl.when(sub == 0)
    def _():
        pltpu.sync_copy(
            contrib_v.at[pl.ds(0, _ZREM2)],
            shared.at[pl.ds(_NS * _ZROW2, _ZREM2)],
        )

    def build(b, _):
        kbase = b * _LANES
        gvec = idx_v[pl.ds(kbase, _LANES)]
        ci_v[pl.ds(kbase, _LANES)] = lax.shift_right_logical(gvec, 7)
        wvec = w_v[pl.ds(kbase, _LANES)]
        rows = kbase + lane
        cols = gvec & (_ROWW - 1)
        plsc.store_scatter(contrib_v, [rows, cols], wvec)
        return 0

    lax.fori_loop(0, _CHUNK // _LANES, build, 0)

    plsc.subcore_barrier()
    pltpu.sync_copy(contrib_v, shared.at[ci_v], add=True)
    plsc.subcore_barrier()

    pltpu.sync_copy(
        shared.at[pl.ds(sub * _ZROW2, _ZROW2)],
        dense_hbm.at[core, pl.ds(sub * _ZROW2, _ZROW2)],
    )

    @pl.when(sub == 0)
    def _():
        pltpu.sync_copy(
            shared.at[pl.ds(_NS * _ZROW2, _ZREM2)],
            dense_hbm.at[core, pl.ds(_NS * _ZROW2, _ZREM2)],
        )


def _mv_body(w_ref, t_ref, o_ref):
    wsum = w_ref[0, :] + w_ref[1, :]

    @pl.when(pl.program_id(0) == 0)
    def _():
        o_ref[...] = jnp.zeros_like(o_ref)

    o_ref[...] += jax.lax.dot_general(
        t_ref[...], wsum,
        dimension_numbers=(((1,), (0,)), ((), ())),
        preferred_element_type=jnp.float32,
    )


_matvec = pl.pallas_call(
    _mv_body,
    grid=(_NSTEP,),
    in_specs=[
        pl.BlockSpec((_NC, _C), lambda i: (0, i)),
        pl.BlockSpec((_DIM, _C), lambda i: (0, i)),
    ],
    out_specs=pl.BlockSpec((_DIM,), lambda i: (0,)),
    out_shape=jax.ShapeDtypeStruct((_DIM,), jnp.float32),
)


def _mv_tail_body(w_ref, t_ref, o_ref):
    wsum = w_ref[0, :] + w_ref[1, :]
    o_ref[...] = jax.lax.dot_general(
        t_ref[...], wsum,
        dimension_numbers=(((1,), (0,)), ((), ())),
        preferred_element_type=jnp.float32,
    )


_mv_tail = pl.pallas_call(
    _mv_tail_body,
    out_shape=jax.ShapeDtypeStruct((_DIM,), jnp.float32),
)


@jax.jit
def kernel(gids, cnts, table):
    gids = gids.astype(jnp.int32)
    w2d = _weights(gids.reshape(128, 128), cnts.reshape(128, 128))
    dense = _sc_scatter(
        gids.reshape(_NW, _CHUNK),
        w2d.reshape(_NW, _CHUNK),
    ).reshape(_NC, _NGPAD)
    tt = table.T
    out_main = _matvec(dense, tt)
    out_tail = _mv_tail(
        lax.slice(dense, (0, _ALIGNED), (_NC, _NG)),
        lax.slice(tt, (0, _ALIGNED), (_DIM, _NG)),
    )
    return out_main + out_tail


# speedup vs baseline: 5.0639x; 1.0510x over previous
import functools

import jax
import jax.numpy as jnp
from jax import lax
from jax.experimental import pallas as pl
from jax.experimental.pallas import tpu as pltpu
from jax.experimental.pallas import tpu_sc as plsc

_DIM = 64
_L = 16384
_NG = 1000000
_NC = 2
_NS = 16
_NW = _NC * _NS
_CHUNK = _L // _NW
_LANES = 16
_NROW = _NG // _LANES
_ZROW = 3904
_ZREM = _NROW - _NS * _ZROW
_WBLK = 488
_C = _WBLK * 128
_NSTEP = 16
_ALIGNED = _NSTEP * _C
_ROWW = 128
_NROW2 = 7816
_NGPAD = _NROW2 * _ROWW
_ZROW2 = 488
_ZREM2 = _NROW2 - _NS * _ZROW2


def _weights_body(g_ref, c_ref, w_ref):
    g = g_ref[...]
    c = c_ref[...]
    t = jnp.log1p(jnp.where(g >= 0, c, 0.0))
    w_ref[...] = t * (1.0 / jnp.sum(t))


_weights = pl.pallas_call(
    _weights_body,
    out_shape=jax.ShapeDtypeStruct((128, 128), jnp.float32),
)

_mesh = plsc.VectorSubcoreMesh(
    core_axis_name="c", subcore_axis_name="s", num_cores=_NC, num_subcores=_NS
)


@functools.partial(
    pl.kernel,
    out_type=jax.ShapeDtypeStruct((_NC, _NROW2, _ROWW), jnp.float32),
    mesh=_mesh,
    compiler_params=pltpu.CompilerParams(needs_layout_passes=False),
    scratch_types=[
        pltpu.VMEM((_CHUNK,), jnp.int32),
        pltpu.VMEM((_CHUNK,), jnp.float32),
        pltpu.VMEM((_CHUNK,), jnp.int32),
        pltpu.VMEM((_CHUNK, _ROWW), jnp.float32),
        pltpu.VMEM_SHARED((_NROW2, _ROWW), jnp.float32),
    ],
)
def _sc_scatter(gids_hbm, w_hbm, dense_hbm,
                idx_v, w_v, ci_v, contrib_v, shared):
    core = lax.axis_index("c")
    sub = lax.axis_index("s")
    wid = sub * _NC + core

    pltpu.sync_copy(gids_hbm.at[wid], idx_v)
    pltpu.sync_copy(w_hbm.at[wid], w_v)

    lane = lax.iota(jnp.int32, _LANES)
    zvec = jnp.zeros((_LANES,), jnp.float32)

    def zfill(r, _):
        for c in range(_ROWW // _LANES):
            contrib_v[r, pl.ds(c * _LANES, _LANES)] = zvec
        return 0

    lax.fori_loop(0, _CHUNK, zfill, 0)

    pltpu.sync_copy(
        contrib_v.at[pl.ds(0, _ZROW2)],
        shared.at[pl.ds(sub * _ZROW2, _ZROW2)],
    )

    @pl.when(sub == 0)
    def _():
        pltpu.sync_copy(
            contrib_v.at[pl.ds(0, _ZREM2)],
            shared.at[pl.ds(_NS * _ZROW2, _ZREM2)],
        )

    def build(b, _):
        kbase = b * _LANES
        gvec = idx_v[pl.ds(kbase, _LANES)]
        ci_v[pl.ds(kbase, _LANES)] = lax.shift_right_logical(gvec, 7)
        wvec = w_v[pl.ds(kbase, _LANES)]
        rows = kbase + lane
        cols = gvec & (_ROWW - 1)
        plsc.store_scatter(contrib_v, [rows, cols], wvec)
        return 0

    lax.fori_loop(0, _CHUNK // _LANES, build, 0)

    plsc.subcore_barrier()
    pltpu.sync_copy(contrib_v, shared.at[ci_v], add=True)
    plsc.subcore_barrier()

    pltpu.sync_copy(
        shared.at[pl.ds(sub * _ZROW2, _ZROW2)],
        dense_hbm.at[core, pl.ds(sub * _ZROW2, _ZROW2)],
    )

    @pl.when(sub == 0)
    def _():
        pltpu.sync_copy(
            shared.at[pl.ds(_NS * _ZROW2, _ZREM2)],
            dense_hbm.at[core, pl.ds(_NS * _ZROW2, _ZREM2)],
        )


def _mv_body(w_ref, t_ref, o_ref):
    @pl.when(pl.program_id(0) == 0)
    def _():
        o_ref[...] = jnp.zeros_like(o_ref)

    o_ref[...] += jax.lax.dot_general(
        t_ref[...], w_ref[...].reshape(_C),
        dimension_numbers=(((1,), (0,)), ((), ())),
        preferred_element_type=jnp.float32,
    )


_matvec = pl.pallas_call(
    _mv_body,
    grid=(_NSTEP,),
    in_specs=[
        pl.BlockSpec((_WBLK, _ROWW), lambda i: (i, 0)),
        pl.BlockSpec((_DIM, _C), lambda i: (0, i)),
    ],
    out_specs=pl.BlockSpec((_DIM,), lambda i: (0,)),
    out_shape=jax.ShapeDtypeStruct((_DIM,), jnp.float32),
)


def _mv_tail_body(w_ref, t_ref, o_ref):
    o_ref[...] = jax.lax.dot_general(
        t_ref[...], w_ref[...],
        dimension_numbers=(((1,), (0,)), ((), ())),
        preferred_element_type=jnp.float32,
    )


_mv_tail = pl.pallas_call(
    _mv_tail_body,
    out_shape=jax.ShapeDtypeStruct((_DIM,), jnp.float32),
)


@jax.jit
def kernel(gids, cnts, table):
    gids = gids.astype(jnp.int32)
    w2d = _weights(gids.reshape(128, 128), cnts.reshape(128, 128))
    dense = _sc_scatter(
        gids.reshape(_NW, _CHUNK),
        w2d.reshape(_NW, _CHUNK),
    )
    wsum = dense[0] + dense[1]
    tt = table.T
    out_main = _matvec(wsum, tt)
    out_tail = _mv_tail(
        lax.slice(wsum.reshape(_NGPAD), (_ALIGNED,), (_NG,)),
        lax.slice(tt, (0, _ALIGNED), (_DIM, _NG)),
    )
    return out_main + out_tail
